# initial kernel scaffold (unmeasured)
import jax
import jax.numpy as jnp
from jax import lax
from jax.experimental import pallas as pl
from jax.experimental.pallas import tpu as pltpu


def kernel(
    x,
):
    def body(*refs):
        pass

    out_shape = jax.ShapeDtypeStruct(..., jnp.float32)
    return pl.pallas_call(body, out_shape=out_shape)(...)


# baseline (device time: 41358 ns/iter reference)
import jax
import jax.numpy as jnp
from jax import lax
from jax.experimental import pallas as pl
from jax.experimental.pallas import tpu as pltpu

N_DEV = 16
M = 512
N = 512
H0 = M // 2


def kernel(x):
    def body(x_ref, out_ref, recv_ref, send_sems, recv_sems):
        i = lax.axis_index("i")
        base = (i // 4) * 4
        s = i % 4
        l = base + (s ^ (s >> 1))

        def to_mesh(lbl):
            sl = lbl % 4
            return (lbl // 4) * 4 + (sl ^ (sl >> 1))

        partners = [to_mesh(l ^ (1 << k)) for k in range(4)]
        bits = [(l >> k) & 1 for k in range(4)]

        barrier_sem = pltpu.get_barrier_semaphore()
        for p in partners:
            pl.semaphore_signal(
                barrier_sem, inc=1,
                device_id=(p,), device_id_type=pl.DeviceIdType.MESH,
            )
        pl.semaphore_wait(barrier_sem, 4)

        out_ref[:, :] = x_ref[0, :, :]

        recv_off = [0, 256, 384, 448]
        lo = 0
        for k in range(4):
            h = H0 >> k
            give_lo = lo + (1 - bits[k]) * h
            keep_lo = lo + bits[k] * h
            rdma = pltpu.make_async_remote_copy(
                src_ref=out_ref.at[pl.ds(give_lo, h)],
                dst_ref=recv_ref.at[pl.ds(recv_off[k], h)],
                send_sem=send_sems.at[k],
                recv_sem=recv_sems.at[k],
                device_id=(partners[k],),
                device_id_type=pl.DeviceIdType.MESH,
            )
            rdma.start()
            rdma.wait()
            out_ref[pl.ds(keep_lo, h), :] = (
                out_ref[pl.ds(keep_lo, h), :] + recv_ref[pl.ds(recv_off[k], h), :]
            )
            lo = keep_lo

        for k in range(3, -1, -1):
            h = H0 >> k
            rdma = pltpu.make_async_remote_copy(
                src_ref=out_ref.at[pl.ds(lo, h)],
                dst_ref=out_ref.at[pl.ds(lo, h)],
                send_sem=send_sems.at[4 + k],
                recv_sem=recv_sems.at[4 + k],
                device_id=(partners[k],),
                device_id_type=pl.DeviceIdType.MESH,
            )
            rdma.start()
            rdma.wait()
            lo = lo - bits[k] * h

    return pl.pallas_call(
        body,
        out_shape=jax.ShapeDtypeStruct((M, N), jnp.float32),
        in_specs=[pl.BlockSpec(memory_space=pltpu.VMEM)],
        out_specs=pl.BlockSpec(memory_space=pltpu.VMEM),
        scratch_shapes=[
            pltpu.VMEM((480, N), jnp.float32),
            pltpu.SemaphoreType.DMA((8,)),
            pltpu.SemaphoreType.DMA((8,)),
        ],
        compiler_params=pltpu.CompilerParams(collective_id=0),
    )(x)


# device time: 32492 ns/iter; 1.2729x vs baseline; 1.2729x over previous
import jax
import jax.numpy as jnp
from jax import lax
from jax.experimental import pallas as pl
from jax.experimental.pallas import tpu as pltpu

N_DEV = 16
M = 512
N = 512
HS = 128


def kernel(x):
    def body(x_ref, out_ref, recv_ref, send_sems, recv_sems):
        i = lax.axis_index("i")
        base = (i // 4) * 4
        s = i % 4
        l = base + (s ^ (s >> 1))

        def to_mesh(lbl):
            sl = lbl % 4
            return (lbl // 4) * 4 + (sl ^ (sl >> 1))

        partner_of = [to_mesh(l ^ (1 << b)) for b in range(4)]
        bit_of = [(l >> b) & 1 for b in range(4)]

        streams = [
            (0, (0, 2, 1, 3), 0, (0, 128, 192, 224)),
            (256, (2, 0, 3, 1), 8, (240, 368, 432, 464)),
        ]

        barrier_sem = pltpu.get_barrier_semaphore()
        for p in partner_of:
            pl.semaphore_signal(
                barrier_sem, inc=1,
                device_id=(p,), device_id_type=pl.DeviceIdType.MESH,
            )
        pl.semaphore_wait(barrier_sem, 4)

        los = [b for b, _, _, _ in streams]
        done = []

        for j in range(4):
            h = HS >> j
            started = []
            for si, (_, order, sem0, roffs) in enumerate(streams):
                b = bit_of[order[j]]
                give_lo = los[si] + (1 - b) * h
                keep_lo = los[si] + b * h
                if j == 0:
                    src = x_ref.at[0, pl.ds(give_lo, h)]
                else:
                    src = out_ref.at[pl.ds(give_lo, h)]
                rdma = pltpu.make_async_remote_copy(
                    src_ref=src,
                    dst_ref=recv_ref.at[pl.ds(roffs[j], h)],
                    send_sem=send_sems.at[sem0 + j],
                    recv_sem=recv_sems.at[sem0 + j],
                    device_id=(partner_of[order[j]],),
                    device_id_type=pl.DeviceIdType.MESH,
                )
                rdma.start()
                started.append((si, rdma, keep_lo, roffs[j]))
            for si, rdma, keep_lo, roff in started:
                rdma.wait_recv()
                if j == 0:
                    mine = x_ref[0, pl.ds(keep_lo, h), :]
                else:
                    mine = out_ref[pl.ds(keep_lo, h), :]
                out_ref[pl.ds(keep_lo, h), :] = mine + recv_ref[pl.ds(roff, h), :]
                los[si] = keep_lo
                done.append(rdma)

        for j in range(3, -1, -1):
            h = HS >> j
            started = []
            for si, (_, order, sem0, _) in enumerate(streams):
                rdma = pltpu.make_async_remote_copy(
                    src_ref=out_ref.at[pl.ds(los[si], h)],
                    dst_ref=out_ref.at[pl.ds(los[si], h)],
                    send_sem=send_sems.at[sem0 + 4 + j],
                    recv_sem=recv_sems.at[sem0 + 4 + j],
                    device_id=(partner_of[order[j]],),
                    device_id_type=pl.DeviceIdType.MESH,
                )
                rdma.start()
                started.append((si, rdma))
            for si, rdma in started:
                rdma.wait_recv()
                los[si] = los[si] - bit_of[streams[si][1][j]] * h
                done.append(rdma)

        for rdma in done:
            rdma.wait_send()

    return pl.pallas_call(
        body,
        out_shape=jax.ShapeDtypeStruct((M, N), jnp.float32),
        in_specs=[pl.BlockSpec(memory_space=pltpu.VMEM)],
        out_specs=pl.BlockSpec(memory_space=pltpu.VMEM),
        scratch_shapes=[
            pltpu.VMEM((480, N), jnp.float32),
            pltpu.SemaphoreType.DMA((16,)),
            pltpu.SemaphoreType.DMA((16,)),
        ],
        compiler_params=pltpu.CompilerParams(collective_id=0),
    )(x)


# device time: 29625 ns/iter; 1.3961x vs baseline; 1.0968x over previous
import jax
import jax.numpy as jnp
from jax import lax
from jax.experimental import pallas as pl
from jax.experimental.pallas import tpu as pltpu

N_DEV = 16
M = 512
N = 512
HS = 64

_STREAMS = (
    (0, (0, 1, 2, 3), 0, (0, 64, 96, 112)),
    (128, (1, 0, 3, 2), 8, (120, 184, 216, 232)),
    (256, (2, 3, 0, 1), 16, (240, 304, 336, 352)),
    (384, (3, 2, 1, 0), 24, (360, 424, 456, 472)),
)


def kernel(x):
    def body(x_ref, out_ref, recv_ref, send_sems, recv_sems):
        i = lax.axis_index("i")
        base = (i // 4) * 4
        s = i % 4
        l = base + (s ^ (s >> 1))

        def to_mesh(lbl):
            sl = lbl % 4
            return (lbl // 4) * 4 + (sl ^ (sl >> 1))

        partner_of = [to_mesh(l ^ (1 << b)) for b in range(4)]
        bit_of = [(l >> b) & 1 for b in range(4)]

        barrier_sem = pltpu.get_barrier_semaphore()
        for p in partner_of:
            pl.semaphore_signal(
                barrier_sem, inc=1,
                device_id=(p,), device_id_type=pl.DeviceIdType.MESH,
            )
        pl.semaphore_wait(barrier_sem, 4)

        los = [st[0] for st in _STREAMS]
        done = []

        def start_slot(si, t):
            base_row, order, sem0, roffs = _STREAMS[si]
            if t < 4:
                j = t
                h = HS >> j
                b = bit_of[order[j]]
                give_lo = los[si] + (1 - b) * h
                keep_lo = los[si] + b * h
                if j == 0:
                    src = x_ref.at[0, pl.ds(give_lo, h)]
                else:
                    src = out_ref.at[pl.ds(give_lo, h)]
                rdma = pltpu.make_async_remote_copy(
                    src_ref=src,
                    dst_ref=recv_ref.at[pl.ds(roffs[j], h)],
                    send_sem=send_sems.at[sem0 + j],
                    recv_sem=recv_sems.at[sem0 + j],
                    device_id=(partner_of[order[j]],),
                    device_id_type=pl.DeviceIdType.MESH,
                )
                rdma.start()
                return (rdma, keep_lo)
            j = 7 - t
            h = HS >> j
            rdma = pltpu.make_async_remote_copy(
                src_ref=out_ref.at[pl.ds(los[si], h)],
                dst_ref=out_ref.at[pl.ds(los[si], h)],
                send_sem=send_sems.at[sem0 + 4 + j],
                recv_sem=recv_sems.at[sem0 + 4 + j],
                device_id=(partner_of[order[j]],),
                device_id_type=pl.DeviceIdType.MESH,
            )
            rdma.start()
            return (rdma, None)

        def finish_slot(si, t, pending):
            rdma, keep_lo = pending
            _, order, _, roffs = _STREAMS[si]
            rdma.wait_recv()
            if t < 4:
                j = t
                h = HS >> j
                if j == 0:
                    mine = x_ref[0, pl.ds(keep_lo, h), :]
                else:
                    mine = out_ref[pl.ds(keep_lo, h), :]
                out_ref[pl.ds(keep_lo, h), :] = (
                    mine + recv_ref[pl.ds(roffs[j], h), :]
                )
                los[si] = keep_lo
            else:
                j = 7 - t
                los[si] = los[si] - bit_of[order[j]] * (HS >> j)
            done.append(rdma)

        n_s = len(_STREAMS)
        pending = [start_slot(si, 0) for si in range(n_s)]
        for t in range(8):
            for si in range(n_s):
                finish_slot(si, t, pending[si])
                if t < 7:
                    pending[si] = start_slot(si, t + 1)

        for rdma in done:
            rdma.wait_send()

    return pl.pallas_call(
        body,
        out_shape=jax.ShapeDtypeStruct((M, N), jnp.float32),
        in_specs=[pl.BlockSpec(memory_space=pltpu.VMEM)],
        out_specs=pl.BlockSpec(memory_space=pltpu.VMEM),
        scratch_shapes=[
            pltpu.VMEM((480, N), jnp.float32),
            pltpu.SemaphoreType.DMA((32,)),
            pltpu.SemaphoreType.DMA((32,)),
        ],
        compiler_params=pltpu.CompilerParams(collective_id=0),
    )(x)


# device time: 26000 ns/iter; 1.5907x vs baseline; 1.1394x over previous
import jax
import jax.numpy as jnp
from jax import lax
from jax.experimental import pallas as pl
from jax.experimental.pallas import tpu as pltpu

N_DEV = 16
M = 512
N = 512
HS = 64

_STREAMS = (
    (0, (0, 1, 2, 3), 0, (0, 64, 96, 128)),
    (128, (1, 0, 3, 2), 6, (160, 224, 256, 288)),
    (256, (2, 3, 0, 1), 12, (320, 384, 416, 448)),
    (384, (3, 2, 1, 0), 18, (480, 544, 576, 608)),
)
_SLOT_H = (64, 32, 32, 32, 32, 64)


def kernel(x):
    def body(x_ref, out_ref, recv_ref, send_sems, recv_sems):
        i = lax.axis_index("i")
        base = (i // 4) * 4
        s = i % 4
        l = base + (s ^ (s >> 1))

        def to_mesh(lbl):
            sl = lbl % 4
            return (lbl // 4) * 4 + (sl ^ (sl >> 1))

        partner_of = [to_mesh(l ^ (1 << b)) for b in range(4)]
        bit_of = [(l >> b) & 1 for b in range(4)]

        barrier_sem = pltpu.get_barrier_semaphore()
        for p in partner_of:
            pl.semaphore_signal(
                barrier_sem, inc=1,
                device_id=(p,), device_id_type=pl.DeviceIdType.MESH,
            )
        pl.semaphore_wait(barrier_sem, 4)

        los = [st[0] for st in _STREAMS]
        done = []

        slot_bit = (0, 1, 2, 3, 1, 0)

        def start_slot(si, t):
            _, order, sem0, roffs = _STREAMS[si]
            h = _SLOT_H[t]
            b = bit_of[order[slot_bit[t]]]
            if t < 2:
                give_lo = los[si] + (1 - b) * h
                if t == 0:
                    src = x_ref.at[0, pl.ds(give_lo, h)]
                else:
                    src = out_ref.at[pl.ds(give_lo, h)]
                dst = recv_ref.at[pl.ds(roffs[t], h)]
                keep_lo = los[si] + b * h
            elif t < 4:
                src = out_ref.at[pl.ds(los[si], h)]
                dst = recv_ref.at[pl.ds(roffs[t], h)]
                keep_lo = los[si]
            else:
                src = out_ref.at[pl.ds(los[si], h)]
                dst = out_ref.at[pl.ds(los[si], h)]
                keep_lo = los[si]
            rdma = pltpu.make_async_remote_copy(
                src_ref=src,
                dst_ref=dst,
                send_sem=send_sems.at[sem0 + t],
                recv_sem=recv_sems.at[sem0 + t],
                device_id=(partner_of[order[slot_bit[t]]],),
                device_id_type=pl.DeviceIdType.MESH,
            )
            rdma.start()
            return (rdma, keep_lo)

        def finish_slot(si, t, pending):
            rdma, keep_lo = pending
            _, order, _, roffs = _STREAMS[si]
            h = _SLOT_H[t]
            b = bit_of[order[slot_bit[t]]]
            if t < 2:
                rdma.wait_recv()
                if t == 0:
                    mine = x_ref[0, pl.ds(keep_lo, h), :]
                else:
                    mine = out_ref[pl.ds(keep_lo, h), :]
                out_ref[pl.ds(keep_lo, h), :] = (
                    mine + recv_ref[pl.ds(roffs[t], h), :]
                )
                los[si] = keep_lo
                done.append(rdma)
            elif t < 4:
                rdma.wait()
                out_ref[pl.ds(keep_lo, h), :] = (
                    out_ref[pl.ds(keep_lo, h), :] + recv_ref[pl.ds(roffs[t], h), :]
                )
            else:
                rdma.wait_recv()
                los[si] = los[si] - b * h
                done.append(rdma)

        n_s = len(_STREAMS)
        pending = [start_slot(si, 0) for si in range(n_s)]
        for t in range(6):
            for si in range(n_s):
                finish_slot(si, t, pending[si])
                if t < 5:
                    pending[si] = start_slot(si, t + 1)

        for rdma in done:
            rdma.wait_send()

    return pl.pallas_call(
        body,
        out_shape=jax.ShapeDtypeStruct((M, N), jnp.float32),
        in_specs=[pl.BlockSpec(memory_space=pltpu.VMEM)],
        out_specs=pl.BlockSpec(memory_space=pltpu.VMEM),
        scratch_shapes=[
            pltpu.VMEM((640, N), jnp.float32),
            pltpu.SemaphoreType.DMA((24,)),
            pltpu.SemaphoreType.DMA((24,)),
        ],
        compiler_params=pltpu.CompilerParams(collective_id=0),
    )(x)
